# baseline (speedup 1.0000x reference)
import functools

import jax
import jax.numpy as jnp
from jax.experimental import pallas as pl
from jax.experimental.pallas import tpu as pltpu

TEMP = 0.05
NUM_CAMS = 8
CC = 2048
RT = 160
NEG = -1e30


def _sorted_ce_kernel(roff_ref, nrt_ref, vlen_ref,
                      xs_ref, tg_ref, tv_ref, vr_ref, p_ref, out_ref,
                      m_ref, s_ref, xn_ref, td_ref,
                      *, nch, bp, b, d):
    k = pl.program_id(0)

    @pl.when(k == 0)
    def _init():
        m_ref[...] = jnp.full((bp, 1), NEG, dtype=jnp.float32)
        s_ref[...] = jnp.zeros((bp, 1), dtype=jnp.float32)
        x = xs_ref[...]
        norm = jnp.sqrt(jnp.sum(x * x, axis=1, keepdims=True))
        xnf = x / (jnp.maximum(norm, 1e-12) * TEMP)
        xn_ref[...] = xnf.astype(jnp.bfloat16)
        td_ref[...] = (jnp.sum(xnf * tg_ref[...], axis=1, keepdims=True)
                       * tv_ref[...])

    p = p_ref[...].astype(jnp.bfloat16)
    vl = vlen_ref[k]
    lane = jax.lax.broadcasted_iota(jnp.int32, (1, CC), 1)
    colpad = lane >= vl
    roff = roff_ref[k]

    def body(t, _):
        ro = pl.multiple_of(roff + t * RT, RT)
        xnb = xn_ref[pl.ds(ro, RT), :]
        sims = jax.lax.dot_general(
            xnb, p, (((1,), (1,)), ((), ())),
            preferred_element_type=jnp.float32)
        msk = jnp.where(colpad, NEG, sims)
        m_old = m_ref[pl.ds(ro, RT), :]
        m_new = jnp.maximum(m_old, jnp.max(msk, axis=1, keepdims=True))
        s_ref[pl.ds(ro, RT), :] = (
            s_ref[pl.ds(ro, RT), :] * jnp.exp(m_old - m_new)
            + jnp.sum(jnp.exp(msk - m_new), axis=1, keepdims=True))
        m_ref[pl.ds(ro, RT), :] = m_new
        return 0

    jax.lax.fori_loop(0, nrt_ref[k], body, 0)

    @pl.when(k == nch - 1)
    def _fin():
        s = s_ref[...]
        per = jnp.where(s > 0.0,
                        m_ref[...] + jnp.log(s) - td_ref[...], 0.0)
        out_ref[...] = jnp.sum(per * vr_ref[...], axis=0,
                               keepdims=True) / b


def _ceil_to(x, m):
    return (x + m - 1) // m * m


def kernel(inputs, targets, cams, proxy, pids, cids):
    del pids
    b, d = inputs.shape
    s = proxy.shape[0]
    nch = -(-s // CC) + NUM_CAMS
    tot = nch * CC
    bp = _ceil_to(b + NUM_CAMS * (RT - 1), RT)

    cid32 = cids.astype(jnp.int32)
    cam32 = cams.astype(jnp.int32)
    t32 = targets.astype(jnp.int32)
    arange8 = jnp.arange(NUM_CAMS, dtype=jnp.int32)

    eqs = (cid32[None, :] == arange8[:, None]).astype(jnp.int32)
    incs = jnp.cumsum(eqs, axis=1)
    counts = incs[:, -1]
    ranks = jnp.sum(eqs * incs, axis=0) - 1
    capc = _ceil_to(counts, CC)
    cumcap = jnp.cumsum(capc)
    cstart = cumcap - capc
    dst = jnp.take(cstart, cid32) + ranks
    perm = jnp.full((tot,), s, dtype=jnp.int32).at[dst].set(
        jnp.arange(s, dtype=jnp.int32), unique_indices=True)

    kcc = jnp.arange(nch, dtype=jnp.int32) * CC
    chunkcam = jnp.clip(jnp.searchsorted(cumcap, kcc, side="right"),
                        0, NUM_CAMS - 1).astype(jnp.int32)
    vlen = jnp.clip(jnp.take(counts, chunkcam)
                    - (kcc - jnp.take(cstart, chunkcam)), 0, CC)

    eqb = (cam32[None, :] == arange8[:, None]).astype(jnp.int32)
    incb = jnp.cumsum(eqb, axis=1)
    countb = incb[:, -1]
    rrank = jnp.sum(eqb * incb, axis=0) - 1
    rcap = _ceil_to(countb, RT)
    rcum = jnp.cumsum(rcap)
    rstart = rcum - rcap
    rdst = jnp.take(rstart, cam32) + rrank
    rowperm = jnp.full((bp,), b, dtype=jnp.int32).at[rdst].set(
        jnp.arange(b, dtype=jnp.int32), unique_indices=True)
    validrow = (rowperm < b).astype(jnp.float32).reshape(bp, 1)

    roff = jnp.take(rstart, chunkcam).astype(jnp.int32)
    nrt = jnp.where(vlen > 0, jnp.take(rcap, chunkcam) // RT,
                    0).astype(jnp.int32)
    vlen = vlen.astype(jnp.int32)

    proxy_p = jnp.pad(proxy, ((0, 1), (0, 0)))
    inputs_p = jnp.pad(inputs, ((0, 1), (0, 0)))
    cam_p = jnp.pad(cam32, (0, 1))
    t_p = jnp.pad(t32, (0, 1))
    camr = jnp.take(cam_p, rowperm)
    tr = jnp.take(t_p, rowperm)
    countr = jnp.take(counts, camr)
    tvalid = (tr < countr) & (rowperm < b)
    tpos = jnp.clip(jnp.take(cstart, camr) + tr, 0, tot - 1)
    pidx = jnp.where(tvalid, jnp.take(perm, tpos), s)

    ps = jnp.take(proxy_p, perm, axis=0)
    xsb = jnp.take(inputs_p, rowperm, axis=0)
    tg = jnp.take(proxy_p, pidx, axis=0)
    tvf = tvalid.astype(jnp.float32).reshape(bp, 1)

    grid_spec = pltpu.PrefetchScalarGridSpec(
        num_scalar_prefetch=3,
        grid=(nch,),
        in_specs=[
            pl.BlockSpec((bp, d), lambda k, *_: (0, 0)),
            pl.BlockSpec((bp, d), lambda k, *_: (0, 0)),
            pl.BlockSpec((bp, 1), lambda k, *_: (0, 0)),
            pl.BlockSpec((bp, 1), lambda k, *_: (0, 0)),
            pl.BlockSpec((CC, d), lambda k, *_: (k, 0)),
        ],
        out_specs=pl.BlockSpec((1, 1), lambda k, *_: (0, 0)),
        scratch_shapes=[
            pltpu.VMEM((bp, 1), jnp.float32),
            pltpu.VMEM((bp, 1), jnp.float32),
            pltpu.VMEM((bp, d), jnp.bfloat16),
            pltpu.VMEM((bp, 1), jnp.float32),
        ],
    )
    out = pl.pallas_call(
        functools.partial(_sorted_ce_kernel, nch=nch, bp=bp, b=b, d=d),
        grid_spec=grid_spec,
        out_shape=jax.ShapeDtypeStruct((1, 1), jnp.float32),
    )(roff, nrt, vlen, xsb, tg, tvf, validrow, ps)
    return out.reshape(1)


# speedup vs baseline: 1.8061x; 1.8061x over previous
import functools

import jax
import jax.numpy as jnp
from jax.experimental import pallas as pl
from jax.experimental.pallas import tpu as pltpu

TEMP = 0.05
NUM_CAMS = 8
CHUNK = 2048
RANK_BITS = 17
NEG = -1e30


def _cam_ce_kernel(x_ref, cams_ref, tg_ref, tv_ref, p_ref, cid_ref, out_ref,
                   m_ref, s_ref, t_ref, xn_ref, oh_ref,
                   *, num_chunks, chunk, b):
    k = pl.program_id(0)

    @pl.when(k == 0)
    def _init():
        m_ref[...] = jnp.full((b, 1), NEG, dtype=jnp.float32)
        s_ref[...] = jnp.zeros((b, 1), dtype=jnp.float32)
        x = x_ref[...]
        norm = jnp.sqrt(jnp.sum(x * x, axis=1, keepdims=True))
        xnf = x / (jnp.maximum(norm, 1e-12) * TEMP)
        xn_ref[...] = xnf.astype(jnp.bfloat16)
        t_ref[...] = (jnp.sum(xnf * tg_ref[...], axis=1, keepdims=True)
                      * tv_ref[...])
        ci = jax.lax.broadcasted_iota(jnp.int32, (b, NUM_CAMS), 1)
        oh_ref[...] = (cams_ref[...] == ci).astype(jnp.float32)

    p = p_ref[...].astype(jnp.bfloat16)
    sims = jax.lax.dot_general(
        xn_ref[...], p, (((1,), (1,)), ((), ())),
        preferred_element_type=jnp.float32)

    cid = cid_ref[0]
    cam_iota = jax.lax.broadcasted_iota(jnp.int32, (NUM_CAMS, chunk), 0)
    eqh = (cid == cam_iota).astype(jnp.bfloat16)

    m_old = m_ref[...]
    m_new = jnp.maximum(m_old, jnp.max(sims, axis=1, keepdims=True))
    e = jnp.exp(sims - m_new).astype(jnp.bfloat16)
    e8 = jax.lax.dot_general(
        e, eqh, (((1,), (1,)), ((), ())),
        preferred_element_type=jnp.float32)
    s_ref[...] = (s_ref[...] * jnp.exp(m_old - m_new)
                  + jnp.sum(e8 * oh_ref[...], axis=1, keepdims=True))
    m_ref[...] = m_new

    @pl.when(k == num_chunks - 1)
    def _fin():
        s = s_ref[...]
        per = jnp.where(s > 0.0,
                        m_ref[...] + jnp.log(s) - t_ref[...], 0.0)
        out_ref[...] = jnp.sum(per, axis=0, keepdims=True) / b


def kernel(inputs, targets, cams, proxy, pids, cids):
    del pids
    b, d = inputs.shape
    s = proxy.shape[0]
    num_chunks = -(-s // CHUNK)
    spad = num_chunks * CHUNK
    proxy_p = jnp.pad(proxy, ((0, 1), (0, 0)))
    cid32 = cids.astype(jnp.int32)
    cam32 = cams.astype(jnp.int32)
    t32 = targets.astype(jnp.int32)

    arange8 = jnp.arange(NUM_CAMS, dtype=jnp.int32)
    eqs = (cid32[None, :] == arange8[:, None]).astype(jnp.int32)
    incs = jnp.cumsum(eqs, axis=1)
    counts = incs[:, -1]
    ranks = jnp.sum(eqs * incs, axis=0) - 1
    key = (cid32 << RANK_BITS) + ranks
    inv = jnp.full((NUM_CAMS << RANK_BITS,), s, dtype=jnp.int32)
    inv = inv.at[key].set(jnp.arange(s, dtype=jnp.int32),
                          unique_indices=True)
    tvalid = t32 < jnp.take(counts, cam32)
    tkey = (cam32 << RANK_BITS) + jnp.clip(t32, 0, (1 << RANK_BITS) - 1)
    pidx = jnp.where(tvalid, jnp.take(inv, tkey), s)
    tg = jnp.take(proxy_p, pidx, axis=0)
    tvf = tvalid.astype(jnp.float32).reshape(b, 1)

    cids_p = jnp.pad(cid32, (0, spad - s), constant_values=NUM_CAMS)
    cids3 = cids_p.reshape(num_chunks, 1, CHUNK)
    proxy_pp = jnp.pad(proxy, ((0, spad - s), (0, 0)))
    cams2 = cam32.reshape(b, 1)

    out = pl.pallas_call(
        functools.partial(_cam_ce_kernel, num_chunks=num_chunks,
                          chunk=CHUNK, b=b),
        grid=(num_chunks,),
        in_specs=[
            pl.BlockSpec((b, d), lambda k: (0, 0)),
            pl.BlockSpec((b, 1), lambda k: (0, 0)),
            pl.BlockSpec((b, d), lambda k: (0, 0)),
            pl.BlockSpec((b, 1), lambda k: (0, 0)),
            pl.BlockSpec((CHUNK, d), lambda k: (k, 0)),
            pl.BlockSpec((1, 1, CHUNK), lambda k: (k, 0, 0)),
        ],
        out_specs=pl.BlockSpec((1, 1), lambda k: (0, 0)),
        out_shape=jax.ShapeDtypeStruct((1, 1), jnp.float32),
        scratch_shapes=[
            pltpu.VMEM((b, 1), jnp.float32),
            pltpu.VMEM((b, 1), jnp.float32),
            pltpu.VMEM((b, 1), jnp.float32),
            pltpu.VMEM((b, d), jnp.bfloat16),
            pltpu.VMEM((b, NUM_CAMS), jnp.float32),
        ],
    )(inputs, cams2, tg, tvf, proxy_pp, cids3)
    return out.reshape(1)


# speedup vs baseline: 5.2480x; 2.9057x over previous
import functools

import jax
import jax.numpy as jnp
from jax.experimental import pallas as pl
from jax.experimental.pallas import tpu as pltpu

TEMP = 0.05
NUM_CAMS = 8
CHUNK = 2048
NEG = -1e30


def _cam_ce_kernel(x_ref, cams_ref, tgt_ref, p_ref, cid_ref, out_ref,
                   m_ref, s_ref, t_ref, c_ref, xn_ref, lt_ref, oh_ref,
                   *, num_chunks, chunk, b):
    k = pl.program_id(0)

    @pl.when(k == 0)
    def _init():
        m_ref[...] = jnp.full((b, 1), NEG, dtype=jnp.float32)
        s_ref[...] = jnp.zeros((b, 1), dtype=jnp.float32)
        t_ref[...] = jnp.zeros((b, 1), dtype=jnp.float32)
        c_ref[...] = jnp.zeros((NUM_CAMS, 1), dtype=jnp.float32)
        x = x_ref[...]
        norm = jnp.sqrt(jnp.sum(x * x, axis=1, keepdims=True))
        xn_ref[...] = (x / (jnp.maximum(norm, 1e-12) * TEMP)
                       ).astype(jnp.bfloat16)
        jj = jax.lax.broadcasted_iota(jnp.int32, (chunk, chunk), 0)
        kk = jax.lax.broadcasted_iota(jnp.int32, (chunk, chunk), 1)
        lt_ref[...] = (jj <= kk).astype(jnp.bfloat16)
        ci = jax.lax.broadcasted_iota(jnp.int32, (b, NUM_CAMS), 1)
        oh_ref[...] = (cams_ref[...] == ci).astype(jnp.float32)

    p = p_ref[...].astype(jnp.bfloat16)
    sims = jax.lax.dot_general(
        xn_ref[...], p, (((1,), (1,)), ((), ())),
        preferred_element_type=jnp.float32)

    cid = cid_ref[0]
    cam_iota = jax.lax.broadcasted_iota(jnp.int32, (NUM_CAMS, chunk), 0)
    eqh = (cid == cam_iota).astype(jnp.bfloat16)
    inc = jax.lax.dot_general(
        eqh, lt_ref[...], (((1,), (0,)), ((), ())),
        preferred_element_type=jnp.float32)
    eq = eqh.astype(jnp.float32)
    base = c_ref[...]
    rank = jnp.sum(eq * (inc + base), axis=0, keepdims=True) - 1.0
    c_ref[...] = base + jnp.sum(eq, axis=1, keepdims=True)

    m_old = m_ref[...]
    m_new = jnp.maximum(m_old, jnp.max(sims, axis=1, keepdims=True))
    e = jnp.exp(sims - m_new).astype(jnp.bfloat16)
    e8 = jax.lax.dot_general(
        e, eqh, (((1,), (1,)), ((), ())),
        preferred_element_type=jnp.float32)
    oh = oh_ref[...]
    s_ref[...] = (s_ref[...] * jnp.exp(m_old - m_new)
                  + jnp.sum(e8 * oh, axis=1, keepdims=True))
    m_ref[...] = m_new

    tf = tgt_ref[...].astype(jnp.float32)
    tmp = jnp.where(rank == tf, sims, 0.0).astype(jnp.bfloat16)
    t8 = jax.lax.dot_general(
        tmp, eqh, (((1,), (1,)), ((), ())),
        preferred_element_type=jnp.float32)
    t_ref[...] = t_ref[...] + jnp.sum(t8 * oh, axis=1, keepdims=True)

    @pl.when(k == num_chunks - 1)
    def _fin():
        s = s_ref[...]
        per = jnp.where(s > 0.0,
                        m_ref[...] + jnp.log(s) - t_ref[...], 0.0)
        out_ref[...] = jnp.sum(per, axis=0, keepdims=True) / b


def kernel(inputs, targets, cams, proxy, pids, cids):
    del pids
    b, d = inputs.shape
    s = proxy.shape[0]
    num_chunks = -(-s // CHUNK)
    spad = num_chunks * CHUNK
    proxy_p = jnp.pad(proxy, ((0, spad - s), (0, 0)))
    cids_p = jnp.pad(cids.astype(jnp.int32), (0, spad - s),
                     constant_values=NUM_CAMS)
    cids3 = cids_p.reshape(num_chunks, 1, CHUNK)
    cams2 = cams.astype(jnp.int32).reshape(b, 1)
    tgts2 = targets.astype(jnp.int32).reshape(b, 1)

    grid = (num_chunks,)
    out = pl.pallas_call(
        functools.partial(_cam_ce_kernel, num_chunks=num_chunks,
                          chunk=CHUNK, b=b),
        grid=grid,
        in_specs=[
            pl.BlockSpec((b, d), lambda k: (0, 0)),
            pl.BlockSpec((b, 1), lambda k: (0, 0)),
            pl.BlockSpec((b, 1), lambda k: (0, 0)),
            pl.BlockSpec((CHUNK, d), lambda k: (k, 0)),
            pl.BlockSpec((1, 1, CHUNK), lambda k: (k, 0, 0)),
        ],
        out_specs=pl.BlockSpec((1, 1), lambda k: (0, 0)),
        out_shape=jax.ShapeDtypeStruct((1, 1), jnp.float32),
        scratch_shapes=[
            pltpu.VMEM((b, 1), jnp.float32),
            pltpu.VMEM((b, 1), jnp.float32),
            pltpu.VMEM((b, 1), jnp.float32),
            pltpu.VMEM((NUM_CAMS, 1), jnp.float32),
            pltpu.VMEM((b, d), jnp.bfloat16),
            pltpu.VMEM((CHUNK, CHUNK), jnp.bfloat16),
            pltpu.VMEM((b, NUM_CAMS), jnp.float32),
        ],
    )(inputs, cams2, tgts2, proxy_p, cids3)
    return out.reshape(1)
